# baseline (speedup 1.0000x reference)
import functools

import jax
import jax.numpy as jnp
from jax import lax
from jax.experimental import pallas as pl
from jax.experimental.pallas import tpu as pltpu
from jax.experimental.pallas import tpu_sc as plsc

N = 10000
E = 320000
C = 128
NET = 7
EPS = 1e-5

NC = 2
NS = 16
NW = NC * NS
CH = C // NC
B = 80
NB = (E // NW) // B
NB2 = (E // NS) // B
G = B // 16
RPT = 1000
NZT = N // RPT
CNT_SZ = N * 8
CHK = 25

_mesh = plsc.VectorSubcoreMesh(core_axis_name="c", subcore_axis_name="s")

_f32 = jnp.float32
_i32 = jnp.int32


@functools.partial(
    pl.kernel,
    out_type=jax.ShapeDtypeStruct((NW, CNT_SZ), _f32),
    mesh=_mesh,
    compiler_params=pltpu.CompilerParams(needs_layout_passes=False),
    scratch_types=[
        pltpu.VMEM((NB, B), _i32),
        pltpu.VMEM((NB, B), _i32),
        pltpu.VMEM((CNT_SZ,), _f32),
    ],
)
def _count_kernel(dst3, et3, zflat, cntp, dst_v, et_v, cnt_v):
  c = lax.axis_index("c")
  s = lax.axis_index("s")
  wid = s * NC + c
  pltpu.sync_copy(dst3.at[wid], dst_v)
  pltpu.sync_copy(et3.at[wid], et_v)
  pltpu.sync_copy(zflat.at[pl.ds(0, CNT_SZ)], cnt_v)
  ones = jnp.ones((16,), _f32)

  def body(b, carry):
    for g in range(G):
      r = dst_v[b, pl.ds(16 * g, 16)]
      e = et_v[b, pl.ds(16 * g, 16)]
      plsc.addupdate_scatter(cnt_v, [r * 8 + e], ones)
    return carry

  lax.fori_loop(0, NB, body, 0)
  pltpu.sync_copy(cnt_v, cntp.at[wid])


@functools.partial(
    pl.kernel,
    out_type=jax.ShapeDtypeStruct((NW, NB, B), _f32),
    mesh=_mesh,
    compiler_params=pltpu.CompilerParams(needs_layout_passes=False),
    scratch_types=[
        pltpu.VMEM((NB, B), _i32),
        pltpu.VMEM((NB, B), _i32),
        pltpu.VMEM((CNT_SZ,), _f32),
        pltpu.VMEM((NB, B), _f32),
    ],
)
def _scale_kernel(dst3, et3, invc, scl3, dst_v, et_v, invc_v, scl_v):
  c = lax.axis_index("c")
  s = lax.axis_index("s")
  wid = s * NC + c
  pltpu.sync_copy(dst3.at[wid], dst_v)
  pltpu.sync_copy(et3.at[wid], et_v)
  pltpu.sync_copy(invc, invc_v)

  def body(b, carry):
    for g in range(G):
      r = dst_v[b, pl.ds(16 * g, 16)]
      e = et_v[b, pl.ds(16 * g, 16)]
      scl_v[b, pl.ds(16 * g, 16)] = plsc.load_gather(invc_v, [r * 8 + e])
    return carry

  lax.fori_loop(0, NB, body, 0)
  pltpu.sync_copy(scl_v, scl3.at[wid])


@functools.partial(
    pl.kernel,
    out_type=jax.ShapeDtypeStruct((NC, N, CH), _f32),
    mesh=_mesh,
    compiler_params=pltpu.CompilerParams(
        needs_layout_passes=False, use_tc_tiling_on_sc=False),
    scratch_types=[
        pltpu.VMEM((NB2, B), _i32),
        pltpu.VMEM((NB2, B), _i32),
        pltpu.VMEM((NB2, B), _f32),
        pltpu.VMEM((CHK, B), _i32),
        pltpu.VMEM((CHK, B), _i32),
        pltpu.VMEM((B, CH), _f32),
        pltpu.VMEM((B, CH), _f32),
        pltpu.VMEM((B, CH), _f32),
        pltpu.VMEM((B, CH), _f32),
        pltpu.VMEM_SHARED((N, CH), _f32),
        pltpu.SemaphoreType.DMA,
        pltpu.SemaphoreType.DMA,
        pltpu.SemaphoreType.DMA,
        pltpu.SemaphoreType.DMA,
    ],
)
def _conv_kernel(zh, col3, et3, dst3, scl3, zeros2d, out,
                 zarr, dst_v, scl_v, ctmp, etmp, rows0, rows1,
                 sc0, sc1, acc, gsem0, gsem1, ssem0, ssem1):
  c = lax.axis_index("c")
  s = lax.axis_index("s")

  @pl.when(s < NZT)
  def _():
    pltpu.sync_copy(zeros2d.at[pl.ds(s * RPT, RPT)],
                    acc.at[pl.ds(s * RPT, RPT)])

  zbase = c * (NET * N)

  for h in range(2):
    pltpu.sync_copy(dst3.at[2 * s + h], dst_v.at[pl.ds(h * NB, NB)])
    pltpu.sync_copy(scl3.at[2 * s + h], scl_v.at[pl.ds(h * NB, NB)])
    for cc in range(NB // CHK):
      pltpu.sync_copy(col3.at[2 * s + h, pl.ds(cc * CHK, CHK)], ctmp)
      pltpu.sync_copy(et3.at[2 * s + h, pl.ds(cc * CHK, CHK)], etmp)
      base_b = h * NB + cc * CHK

      def zbody(g2, carry):
        for g in range(G):
          zarr[base_b + g2, pl.ds(16 * g, 16)] = (
              etmp[g2, pl.ds(16 * g, 16)] * N
              + ctmp[g2, pl.ds(16 * g, 16)] + zbase)
        return carry

      lax.fori_loop(0, CHK, zbody, 0)
  plsc.subcore_barrier()

  def scale_rows(b, rbuf, obuf):
    bvec = jnp.full((16,), b, _i32)
    for i in range(B):
      bc = plsc.load_gather(scl_v, [bvec, jnp.full((16,), i, _i32)])
      for k in range(CH // 16):
        obuf[i, pl.ds(16 * k, 16)] = rbuf[i, pl.ds(16 * k, 16)] * bc

  pltpu.async_copy(zh.at[zarr.at[0]], rows0, gsem0)
  pltpu.async_copy(zh.at[zarr.at[1]], rows1, gsem1)

  npairs = NB2 // 2

  def body(j, carry):
    b0 = 2 * j
    b1 = 2 * j + 1
    pltpu.make_async_copy(zh.at[zarr.at[b0]], rows0, gsem0).wait()
    scale_rows(b0, rows0, sc0)
    d0 = pltpu.async_copy(sc0, acc.at[dst_v.at[b0]], ssem0, add=True)
    pltpu.make_async_copy(zh.at[zarr.at[b1]], rows1, gsem1).wait()
    scale_rows(b1, rows1, sc1)
    d1 = pltpu.async_copy(sc1, acc.at[dst_v.at[b1]], ssem1, add=True)

    @pl.when(j < npairs - 1)
    def _():
      d0.wait()
      pltpu.async_copy(zh.at[zarr.at[b0 + 2]], rows0, gsem0)
      d1.wait()
      pltpu.async_copy(zh.at[zarr.at[b1 + 2]], rows1, gsem1)

    return carry

  lax.fori_loop(0, npairs, body, 0)
  pltpu.make_async_copy(sc0, acc.at[dst_v.at[NB2 - 2]], ssem0).wait()
  pltpu.make_async_copy(sc1, acc.at[dst_v.at[NB2 - 1]], ssem1).wait()
  plsc.subcore_barrier()

  @pl.when(s < NZT)
  def _():
    pltpu.sync_copy(acc.at[pl.ds(s * RPT, RPT)],
                    out.at[c, pl.ds(s * RPT, RPT)])


_BN = 1000
_NBK = N // _BN


def _m1_body(x_ref, w_ref, cntp_ref, z_ref, invc_ref):
  t = pl.program_id(0)
  i = pl.program_id(1)
  z = jnp.dot(x_ref[...], w_ref[0], preferred_element_type=_f32)
  z_ref[0, 0] = z[:, :CH]
  z_ref[1, 0] = z[:, CH:]

  @pl.when(jnp.logical_and(t == 0, i == 0))
  def _():
    csum = jnp.sum(cntp_ref[...], axis=0)
    invc_ref[...] = 1.0 / jnp.maximum(csum, 1.0)


def _z1_and_invc(x, w_r, cntp):
  return pl.pallas_call(
      _m1_body,
      grid=(NET, _NBK),
      in_specs=[
          pl.BlockSpec((_BN, C), lambda t, i: (i, 0)),
          pl.BlockSpec((1, C, C), lambda t, i: (t, 0, 0)),
          pl.BlockSpec((NW, CNT_SZ // C, C), lambda t, i: (0, 0, 0)),
      ],
      out_specs=[
          pl.BlockSpec((NC, 1, _BN, CH), lambda t, i: (0, t, i, 0)),
          pl.BlockSpec((CNT_SZ // C, C), lambda t, i: (0, 0)),
      ],
      out_shape=[
          jax.ShapeDtypeStruct((NC, NET, N, CH), _f32),
          jax.ShapeDtypeStruct((CNT_SZ // C, C), _f32),
      ],
  )(x, w_r, cntp)


def _bn1m2_body(s_ref, wb_ref, ga_ref, ba_ref, z2_ref, sum_ref, sq_ref):
  p = pl.program_id(0)
  i = pl.program_id(1)
  y = jnp.concatenate([s_ref[0], s_ref[1]], axis=-1)

  @pl.when(jnp.logical_and(p == 0, i == 0))
  def _():
    sum_ref[...] = jnp.zeros_like(sum_ref)
    sq_ref[...] = jnp.zeros_like(sq_ref)

  @pl.when(p == 0)
  def _():
    sum_ref[...] += jnp.sum(y, axis=0, keepdims=True)
    sq_ref[...] += jnp.sum(y * y, axis=0, keepdims=True)

  @pl.when(p == 1)
  def _():
    mean = sum_ref[...] * (1.0 / N)
    var = sq_ref[...] * (1.0 / N) - mean * mean
    inv = lax.rsqrt(var + EPS)
    x1 = jnp.maximum((y - mean) * inv * ga_ref[...] + ba_ref[...], 0.0)
    for t in range(NET):
      z = jnp.dot(x1, wb_ref[t], preferred_element_type=_f32)
      z2_ref[0, t] = z[:, :CH]
      z2_ref[1, t] = z[:, CH:]


def _bn1_then_z2(s1, wb_r, ga, ba):
  return pl.pallas_call(
      _bn1m2_body,
      grid=(2, _NBK),
      in_specs=[
          pl.BlockSpec((NC, _BN, CH), lambda p, i: (0, i, 0)),
          pl.BlockSpec((NET, C, C), lambda p, i: (0, 0, 0)),
          pl.BlockSpec((1, C), lambda p, i: (0, 0)),
          pl.BlockSpec((1, C), lambda p, i: (0, 0)),
      ],
      out_specs=pl.BlockSpec((NC, NET, _BN, CH), lambda p, i: (0, 0, i, 0)),
      out_shape=jax.ShapeDtypeStruct((NC, NET, N, CH), _f32),
      scratch_shapes=[
          pltpu.VMEM((1, C), _f32),
          pltpu.VMEM((1, C), _f32),
      ],
  )(s1, wb_r, ga, ba)


def _bn2_body(s_ref, x_ref, gb_ref, bb_ref, o_ref, sum_ref, sq_ref):
  p = pl.program_id(0)
  i = pl.program_id(1)
  y = jnp.concatenate([s_ref[0], s_ref[1]], axis=-1)

  @pl.when(jnp.logical_and(p == 0, i == 0))
  def _():
    sum_ref[...] = jnp.zeros_like(sum_ref)
    sq_ref[...] = jnp.zeros_like(sq_ref)

  @pl.when(p == 0)
  def _():
    sum_ref[...] += jnp.sum(y, axis=0, keepdims=True)
    sq_ref[...] += jnp.sum(y * y, axis=0, keepdims=True)

  @pl.when(p == 1)
  def _():
    mean = sum_ref[...] * (1.0 / N)
    var = sq_ref[...] * (1.0 / N) - mean * mean
    inv = lax.rsqrt(var + EPS)
    o_ref[...] = jnp.maximum(
        (y - mean) * inv * gb_ref[...] + bb_ref[...] + x_ref[...], 0.0)


def _bn2_skip(s2, x, gb, bb):
  return pl.pallas_call(
      _bn2_body,
      grid=(2, _NBK),
      in_specs=[
          pl.BlockSpec((NC, _BN, CH), lambda p, i: (0, i, 0)),
          pl.BlockSpec((_BN, C), lambda p, i: (i, 0)),
          pl.BlockSpec((1, C), lambda p, i: (0, 0)),
          pl.BlockSpec((1, C), lambda p, i: (0, 0)),
      ],
      out_specs=pl.BlockSpec((_BN, C), lambda p, i: (i, 0)),
      out_shape=jax.ShapeDtypeStruct((N, C), _f32),
      scratch_shapes=[
          pltpu.VMEM((1, C), _f32),
          pltpu.VMEM((1, C), _f32),
      ],
  )(s2, x, gb, bb)


def kernel(x, edge_index, edge_type, node_type, Wa, ga, ba, Wb, gb, bb):
  del node_type
  row = edge_index[0]
  col = edge_index[1]
  dst3 = row.reshape(NW, NB, B)
  col3 = col.reshape(NW, NB, B)
  et3 = edge_type.reshape(NW, NB, B)
  zeros2d = jnp.zeros((N, CH), _f32)
  zflat = jnp.zeros((CNT_SZ,), _f32)

  wa_r = Wa.reshape(NET, C, C)
  wb_r = Wb.reshape(NET, C, C)

  cntp = _count_kernel(dst3, et3, zflat)
  z1, invc = _z1_and_invc(x, wa_r, cntp.reshape(NW, CNT_SZ // C, C))
  scl3 = _scale_kernel(dst3, et3, invc.reshape(CNT_SZ))

  s1 = _conv_kernel(z1.reshape(NC * NET * N, CH), col3, et3, dst3, scl3,
                    zeros2d)
  z2 = _bn1_then_z2(s1, wb_r, ga.reshape(1, C), ba.reshape(1, C))
  s2 = _conv_kernel(z2.reshape(NC * NET * N, CH), col3, et3, dst3, scl3,
                    zeros2d)
  return _bn2_skip(s2, x, gb.reshape(1, C), bb.reshape(1, C))


# speedup vs baseline: 1.5871x; 1.5871x over previous
import functools

import jax
import jax.numpy as jnp
from jax import lax
from jax.experimental import pallas as pl
from jax.experimental.pallas import tpu as pltpu
from jax.experimental.pallas import tpu_sc as plsc

N = 10000
E = 320000
C = 128
NET = 7
EPS = 1e-5

NC = 2
NS = 16
NW = NC * NS
CH = C // NC
B = 80
NB = (E // NW) // B
NB2 = (E // NS) // B
G = B // 16
RPT = 1000
NZT = N // RPT
CNT_SZ = N * 8
CHK = 25

_mesh = plsc.VectorSubcoreMesh(core_axis_name="c", subcore_axis_name="s")

_f32 = jnp.float32
_i32 = jnp.int32


@functools.partial(
    pl.kernel,
    out_type=jax.ShapeDtypeStruct((NW, CNT_SZ), _f32),
    mesh=_mesh,
    compiler_params=pltpu.CompilerParams(needs_layout_passes=False),
    scratch_types=[
        pltpu.VMEM((NB, B), _i32),
        pltpu.VMEM((NB, B), _i32),
        pltpu.VMEM((CNT_SZ,), _f32),
    ],
)
def _count_kernel(dst3, et3, zflat, cntp, dst_v, et_v, cnt_v):
  c = lax.axis_index("c")
  s = lax.axis_index("s")
  wid = s * NC + c
  pltpu.sync_copy(dst3.at[wid], dst_v)
  pltpu.sync_copy(et3.at[wid], et_v)
  pltpu.sync_copy(zflat.at[pl.ds(0, CNT_SZ)], cnt_v)
  ones = jnp.ones((16,), _f32)

  def body(b, carry):
    for g in range(G):
      r = dst_v[b, pl.ds(16 * g, 16)]
      e = et_v[b, pl.ds(16 * g, 16)]
      plsc.addupdate_scatter(cnt_v, [r * 8 + e], ones)
    return carry

  lax.fori_loop(0, NB, body, 0)
  pltpu.sync_copy(cnt_v, cntp.at[wid])


@functools.partial(
    pl.kernel,
    out_type=jax.ShapeDtypeStruct((NW, NB, B), _f32),
    mesh=_mesh,
    compiler_params=pltpu.CompilerParams(needs_layout_passes=False),
    scratch_types=[
        pltpu.VMEM((NB, B), _i32),
        pltpu.VMEM((NB, B), _i32),
        pltpu.VMEM((CNT_SZ,), _f32),
        pltpu.VMEM((NB, B), _f32),
    ],
)
def _scale_kernel(dst3, et3, invc, scl3, dst_v, et_v, invc_v, scl_v):
  c = lax.axis_index("c")
  s = lax.axis_index("s")
  wid = s * NC + c
  pltpu.sync_copy(dst3.at[wid], dst_v)
  pltpu.sync_copy(et3.at[wid], et_v)
  pltpu.sync_copy(invc, invc_v)

  def body(b, carry):
    for g in range(G):
      r = dst_v[b, pl.ds(16 * g, 16)]
      e = et_v[b, pl.ds(16 * g, 16)]
      scl_v[b, pl.ds(16 * g, 16)] = plsc.load_gather(invc_v, [r * 8 + e])
    return carry

  lax.fori_loop(0, NB, body, 0)
  pltpu.sync_copy(scl_v, scl3.at[wid])


@functools.partial(
    pl.kernel,
    out_type=jax.ShapeDtypeStruct((NC, N, CH), _f32),
    mesh=_mesh,
    compiler_params=pltpu.CompilerParams(
        needs_layout_passes=False, use_tc_tiling_on_sc=False),
    scratch_types=[
        pltpu.VMEM((NB2, B), _i32),
        pltpu.VMEM((NB2, B), _i32),
        pltpu.VMEM((NB2, B), _f32),
        pltpu.VMEM((CHK, B), _i32),
        pltpu.VMEM((CHK, B), _i32),
        pltpu.VMEM((B, CH), _f32),
        pltpu.VMEM((B, CH), _f32),
        pltpu.VMEM((B, CH), _f32),
        pltpu.VMEM((B, CH), _f32),
        pltpu.VMEM_SHARED((N, CH), _f32),
        pltpu.SemaphoreType.DMA,
        pltpu.SemaphoreType.DMA,
        pltpu.SemaphoreType.DMA,
        pltpu.SemaphoreType.DMA,
    ],
)
def _conv_kernel(zh, col3, et3, dst3, scl3, zeros2d, out,
                 zarr, dst_v, scl_v, ctmp, etmp, rows0, rows1,
                 sc0, sc1, acc, gsem0, gsem1, ssem0, ssem1):
  c = lax.axis_index("c")
  s = lax.axis_index("s")

  @pl.when(s < NZT)
  def _():
    pltpu.sync_copy(zeros2d.at[pl.ds(s * RPT, RPT)],
                    acc.at[pl.ds(s * RPT, RPT)])

  zbase = c * (NET * N)

  for h in range(2):
    pltpu.sync_copy(dst3.at[2 * s + h], dst_v.at[pl.ds(h * NB, NB)])
    pltpu.sync_copy(scl3.at[2 * s + h], scl_v.at[pl.ds(h * NB, NB)])
    for cc in range(NB // CHK):
      pltpu.sync_copy(col3.at[2 * s + h, pl.ds(cc * CHK, CHK)], ctmp)
      pltpu.sync_copy(et3.at[2 * s + h, pl.ds(cc * CHK, CHK)], etmp)
      base_b = h * NB + cc * CHK

      def zbody(g2, carry):
        for g in range(G):
          zarr[base_b + g2, pl.ds(16 * g, 16)] = (
              etmp[g2, pl.ds(16 * g, 16)] * N
              + ctmp[g2, pl.ds(16 * g, 16)] + zbase)
        return carry

      lax.fori_loop(0, CHK, zbody, 0)
  plsc.subcore_barrier()

  def scale_rows(b, rbuf, obuf):
    bvec = jnp.full((16,), b, _i32)

    @plsc.parallel_loop(0, B, 1, unroll=8)
    def _(i):
      bc = plsc.load_gather(scl_v, [bvec, jnp.full((16,), i, _i32)])
      for k in range(CH // 16):
        obuf[i, pl.ds(16 * k, 16)] = rbuf[i, pl.ds(16 * k, 16)] * bc

  pltpu.async_copy(zh.at[zarr.at[0]], rows0, gsem0)
  pltpu.async_copy(zh.at[zarr.at[1]], rows1, gsem1)

  npairs = NB2 // 2

  def body(j, carry):
    b0 = 2 * j
    b1 = 2 * j + 1
    pltpu.make_async_copy(zh.at[zarr.at[b0]], rows0, gsem0).wait()
    scale_rows(b0, rows0, sc0)
    d0 = pltpu.async_copy(sc0, acc.at[dst_v.at[b0]], ssem0, add=True)
    pltpu.make_async_copy(zh.at[zarr.at[b1]], rows1, gsem1).wait()
    scale_rows(b1, rows1, sc1)
    d1 = pltpu.async_copy(sc1, acc.at[dst_v.at[b1]], ssem1, add=True)

    @pl.when(j < npairs - 1)
    def _():
      d0.wait()
      pltpu.async_copy(zh.at[zarr.at[b0 + 2]], rows0, gsem0)
      d1.wait()
      pltpu.async_copy(zh.at[zarr.at[b1 + 2]], rows1, gsem1)

    return carry

  lax.fori_loop(0, npairs, body, 0)
  pltpu.make_async_copy(sc0, acc.at[dst_v.at[NB2 - 2]], ssem0).wait()
  pltpu.make_async_copy(sc1, acc.at[dst_v.at[NB2 - 1]], ssem1).wait()
  plsc.subcore_barrier()

  @pl.when(s < NZT)
  def _():
    pltpu.sync_copy(acc.at[pl.ds(s * RPT, RPT)],
                    out.at[c, pl.ds(s * RPT, RPT)])


_BN = 1000
_NBK = N // _BN


def _m1_body(x_ref, w_ref, cntp_ref, z_ref, invc_ref):
  t = pl.program_id(0)
  i = pl.program_id(1)
  z = jnp.dot(x_ref[...], w_ref[0], preferred_element_type=_f32)
  z_ref[0, 0] = z[:, :CH]
  z_ref[1, 0] = z[:, CH:]

  @pl.when(jnp.logical_and(t == 0, i == 0))
  def _():
    csum = jnp.sum(cntp_ref[...], axis=0)
    invc_ref[...] = 1.0 / jnp.maximum(csum, 1.0)


def _z1_and_invc(x, w_r, cntp):
  return pl.pallas_call(
      _m1_body,
      grid=(NET, _NBK),
      in_specs=[
          pl.BlockSpec((_BN, C), lambda t, i: (i, 0)),
          pl.BlockSpec((1, C, C), lambda t, i: (t, 0, 0)),
          pl.BlockSpec((NW, CNT_SZ // C, C), lambda t, i: (0, 0, 0)),
      ],
      out_specs=[
          pl.BlockSpec((NC, 1, _BN, CH), lambda t, i: (0, t, i, 0)),
          pl.BlockSpec((CNT_SZ // C, C), lambda t, i: (0, 0)),
      ],
      out_shape=[
          jax.ShapeDtypeStruct((NC, NET, N, CH), _f32),
          jax.ShapeDtypeStruct((CNT_SZ // C, C), _f32),
      ],
  )(x, w_r, cntp)


def _bn1m2_body(s_ref, wb_ref, ga_ref, ba_ref, z2_ref, sum_ref, sq_ref):
  p = pl.program_id(0)
  i = pl.program_id(1)
  y = jnp.concatenate([s_ref[0], s_ref[1]], axis=-1)

  @pl.when(jnp.logical_and(p == 0, i == 0))
  def _():
    sum_ref[...] = jnp.zeros_like(sum_ref)
    sq_ref[...] = jnp.zeros_like(sq_ref)

  @pl.when(p == 0)
  def _():
    sum_ref[...] += jnp.sum(y, axis=0, keepdims=True)
    sq_ref[...] += jnp.sum(y * y, axis=0, keepdims=True)

  @pl.when(p == 1)
  def _():
    mean = sum_ref[...] * (1.0 / N)
    var = sq_ref[...] * (1.0 / N) - mean * mean
    inv = lax.rsqrt(var + EPS)
    x1 = jnp.maximum((y - mean) * inv * ga_ref[...] + ba_ref[...], 0.0)
    for t in range(NET):
      z = jnp.dot(x1, wb_ref[t], preferred_element_type=_f32)
      z2_ref[0, t] = z[:, :CH]
      z2_ref[1, t] = z[:, CH:]


def _bn1_then_z2(s1, wb_r, ga, ba):
  return pl.pallas_call(
      _bn1m2_body,
      grid=(2, _NBK),
      in_specs=[
          pl.BlockSpec((NC, _BN, CH), lambda p, i: (0, i, 0)),
          pl.BlockSpec((NET, C, C), lambda p, i: (0, 0, 0)),
          pl.BlockSpec((1, C), lambda p, i: (0, 0)),
          pl.BlockSpec((1, C), lambda p, i: (0, 0)),
      ],
      out_specs=pl.BlockSpec((NC, NET, _BN, CH), lambda p, i: (0, 0, i, 0)),
      out_shape=jax.ShapeDtypeStruct((NC, NET, N, CH), _f32),
      scratch_shapes=[
          pltpu.VMEM((1, C), _f32),
          pltpu.VMEM((1, C), _f32),
      ],
  )(s1, wb_r, ga, ba)


def _bn2_body(s_ref, x_ref, gb_ref, bb_ref, o_ref, sum_ref, sq_ref):
  p = pl.program_id(0)
  i = pl.program_id(1)
  y = jnp.concatenate([s_ref[0], s_ref[1]], axis=-1)

  @pl.when(jnp.logical_and(p == 0, i == 0))
  def _():
    sum_ref[...] = jnp.zeros_like(sum_ref)
    sq_ref[...] = jnp.zeros_like(sq_ref)

  @pl.when(p == 0)
  def _():
    sum_ref[...] += jnp.sum(y, axis=0, keepdims=True)
    sq_ref[...] += jnp.sum(y * y, axis=0, keepdims=True)

  @pl.when(p == 1)
  def _():
    mean = sum_ref[...] * (1.0 / N)
    var = sq_ref[...] * (1.0 / N) - mean * mean
    inv = lax.rsqrt(var + EPS)
    o_ref[...] = jnp.maximum(
        (y - mean) * inv * gb_ref[...] + bb_ref[...] + x_ref[...], 0.0)


def _bn2_skip(s2, x, gb, bb):
  return pl.pallas_call(
      _bn2_body,
      grid=(2, _NBK),
      in_specs=[
          pl.BlockSpec((NC, _BN, CH), lambda p, i: (0, i, 0)),
          pl.BlockSpec((_BN, C), lambda p, i: (i, 0)),
          pl.BlockSpec((1, C), lambda p, i: (0, 0)),
          pl.BlockSpec((1, C), lambda p, i: (0, 0)),
      ],
      out_specs=pl.BlockSpec((_BN, C), lambda p, i: (i, 0)),
      out_shape=jax.ShapeDtypeStruct((N, C), _f32),
      scratch_shapes=[
          pltpu.VMEM((1, C), _f32),
          pltpu.VMEM((1, C), _f32),
      ],
  )(s2, x, gb, bb)


def kernel(x, edge_index, edge_type, node_type, Wa, ga, ba, Wb, gb, bb):
  del node_type
  row = edge_index[0]
  col = edge_index[1]
  dst3 = row.reshape(NW, NB, B)
  col3 = col.reshape(NW, NB, B)
  et3 = edge_type.reshape(NW, NB, B)
  zeros2d = jnp.zeros((N, CH), _f32)
  zflat = jnp.zeros((CNT_SZ,), _f32)

  wa_r = Wa.reshape(NET, C, C)
  wb_r = Wb.reshape(NET, C, C)

  cntp = _count_kernel(dst3, et3, zflat)
  z1, invc = _z1_and_invc(x, wa_r, cntp.reshape(NW, CNT_SZ // C, C))
  scl3 = _scale_kernel(dst3, et3, invc.reshape(CNT_SZ))

  s1 = _conv_kernel(z1.reshape(NC * NET * N, CH), col3, et3, dst3, scl3,
                    zeros2d)
  z2 = _bn1_then_z2(s1, wb_r, ga.reshape(1, C), ba.reshape(1, C))
  s2 = _conv_kernel(z2.reshape(NC * NET * N, CH), col3, et3, dst3, scl3,
                    zeros2d)
  return _bn2_skip(s2, x, gb.reshape(1, C), bb.reshape(1, C))


# speedup vs baseline: 1.9477x; 1.2272x over previous
import functools

import jax
import jax.numpy as jnp
from jax import lax
from jax.experimental import pallas as pl
from jax.experimental.pallas import tpu as pltpu
from jax.experimental.pallas import tpu_sc as plsc

N = 10000
E = 320000
C = 128
NET = 7
EPS = 1e-5

NC = 2
NS = 16
NW = NC * NS
CH = C // NC
B = 80
NB = (E // NW) // B
NB2 = (E // NS) // B
G = B // 16
RPT = 1000
NZT = N // RPT
CNT_SZ = N * 8
CHK = 25

_mesh = plsc.VectorSubcoreMesh(core_axis_name="c", subcore_axis_name="s")

_f32 = jnp.float32
_i32 = jnp.int32


@functools.partial(
    pl.kernel,
    out_type=jax.ShapeDtypeStruct((NW, CNT_SZ), _f32),
    mesh=_mesh,
    compiler_params=pltpu.CompilerParams(needs_layout_passes=False),
    scratch_types=[
        pltpu.VMEM((NB, B), _i32),
        pltpu.VMEM((NB, B), _i32),
        pltpu.VMEM((CNT_SZ,), _f32),
    ],
)
def _count_kernel(dst3, et3, zflat, cntp, dst_v, et_v, cnt_v):
  c = lax.axis_index("c")
  s = lax.axis_index("s")
  wid = s * NC + c
  pltpu.sync_copy(dst3.at[wid], dst_v)
  pltpu.sync_copy(et3.at[wid], et_v)
  pltpu.sync_copy(zflat.at[pl.ds(0, CNT_SZ)], cnt_v)
  ones = jnp.ones((16,), _f32)

  def body(b, carry):
    for g in range(G):
      r = dst_v[b, pl.ds(16 * g, 16)]
      e = et_v[b, pl.ds(16 * g, 16)]
      plsc.addupdate_scatter(cnt_v, [r * 8 + e], ones)
    return carry

  lax.fori_loop(0, NB, body, 0)
  pltpu.sync_copy(cnt_v, cntp.at[wid])


@functools.partial(
    pl.kernel,
    out_type=jax.ShapeDtypeStruct((NW, NB, B), _f32),
    mesh=_mesh,
    compiler_params=pltpu.CompilerParams(needs_layout_passes=False),
    scratch_types=[
        pltpu.VMEM((NB, B), _i32),
        pltpu.VMEM((NB, B), _i32),
        pltpu.VMEM((CNT_SZ,), _f32),
        pltpu.VMEM((NB, B), _f32),
    ],
)
def _scale_kernel(dst3, et3, invc, scl3, dst_v, et_v, invc_v, scl_v):
  c = lax.axis_index("c")
  s = lax.axis_index("s")
  wid = s * NC + c
  pltpu.sync_copy(dst3.at[wid], dst_v)
  pltpu.sync_copy(et3.at[wid], et_v)
  pltpu.sync_copy(invc, invc_v)

  def body(b, carry):
    for g in range(G):
      r = dst_v[b, pl.ds(16 * g, 16)]
      e = et_v[b, pl.ds(16 * g, 16)]
      scl_v[b, pl.ds(16 * g, 16)] = plsc.load_gather(invc_v, [r * 8 + e])
    return carry

  lax.fori_loop(0, NB, body, 0)
  pltpu.sync_copy(scl_v, scl3.at[wid])


@functools.partial(
    pl.kernel,
    out_type=jax.ShapeDtypeStruct((NC, N, CH), _f32),
    mesh=_mesh,
    compiler_params=pltpu.CompilerParams(
        needs_layout_passes=False, use_tc_tiling_on_sc=False),
    scratch_types=[
        pltpu.VMEM((NB2, B), _i32),
        pltpu.VMEM((NB2, B), _i32),
        pltpu.VMEM((NB2, B), _f32),
        pltpu.VMEM((CHK, B), _i32),
        pltpu.VMEM((CHK, B), _i32),
        pltpu.VMEM((B, CH), _f32),
        pltpu.VMEM((B, CH), _f32),
        pltpu.VMEM((B, CH), _f32),
        pltpu.VMEM((B, CH), _f32),
        pltpu.VMEM((B, CH), _f32),
        pltpu.VMEM_SHARED((N, CH), _f32),
        pltpu.SemaphoreType.DMA,
        pltpu.SemaphoreType.DMA,
        pltpu.SemaphoreType.DMA,
        pltpu.SemaphoreType.DMA,
        pltpu.SemaphoreType.DMA,
        pltpu.SemaphoreType.DMA,
        pltpu.SemaphoreType.DMA,
        pltpu.SemaphoreType.DMA,
        pltpu.SemaphoreType.DMA,
        pltpu.SemaphoreType.DMA,
    ],
)
def _conv_kernel(zh, col3, et3, dst3, scl3, zeros2d, out,
                 zarr, dst_v, scl_v, ctmp, etmp,
                 r0, r1, r2, r3, r4, acc,
                 g0, g1, g2, g3, g4, s0, s1, s2, s3, s4):
  c = lax.axis_index("c")
  s = lax.axis_index("s")

  @pl.when(s < NZT)
  def _():
    pltpu.sync_copy(zeros2d.at[pl.ds(s * RPT, RPT)],
                    acc.at[pl.ds(s * RPT, RPT)])

  zbase = c * (NET * N)

  for h in range(2):
    pltpu.sync_copy(dst3.at[2 * s + h], dst_v.at[pl.ds(h * NB, NB)])
    pltpu.sync_copy(scl3.at[2 * s + h], scl_v.at[pl.ds(h * NB, NB)])
    for cc in range(NB // CHK):
      pltpu.sync_copy(col3.at[2 * s + h, pl.ds(cc * CHK, CHK)], ctmp)
      pltpu.sync_copy(et3.at[2 * s + h, pl.ds(cc * CHK, CHK)], etmp)
      base_b = h * NB + cc * CHK

      def zbody(g2, carry):
        for g in range(G):
          zarr[base_b + g2, pl.ds(16 * g, 16)] = (
              etmp[g2, pl.ds(16 * g, 16)] * N
              + ctmp[g2, pl.ds(16 * g, 16)] + zbase)
        return carry

      lax.fori_loop(0, CHK, zbody, 0)
  plsc.subcore_barrier()

  rows = [r0, r1, r2, r3, r4]
  gsem = [g0, g1, g2, g3, g4]
  ssem = [s0, s1, s2, s3, s4]
  NBUF = 5

  def scale_rows(b, rbuf):
    bvec = jnp.full((16,), b, _i32)

    @plsc.parallel_loop(0, B, 1, unroll=8)
    def _(i):
      bc = plsc.load_gather(scl_v, [bvec, jnp.full((16,), i, _i32)])
      for k in range(CH // 16):
        rbuf[i, pl.ds(16 * k, 16)] = rbuf[i, pl.ds(16 * k, 16)] * bc

  for q in range(3):
    pltpu.async_copy(zh.at[zarr.at[q]], rows[q], gsem[q])

  def round_body(j, carry):
    for q in range(NBUF):
      b = NBUF * j + q
      qn = (q + 3) % NBUF
      pltpu.make_async_copy(zh.at[zarr.at[b]], rows[q], gsem[q]).wait()
      scale_rows(b, rows[q])
      pltpu.async_copy(rows[q], acc.at[dst_v.at[b]], ssem[q], add=True)

      @pl.when(b >= 2)
      def _(qn=qn, b=b):
        pltpu.make_async_copy(
            rows[qn], acc.at[dst_v.at[b - 2]], ssem[qn]).wait()

      @pl.when(b + 3 < NB2)
      def _(qn=qn, b=b):
        pltpu.async_copy(zh.at[zarr.at[b + 3]], rows[qn], gsem[qn])

    return carry

  lax.fori_loop(0, NB2 // NBUF, round_body, 0)
  pltpu.make_async_copy(rows[3], acc.at[dst_v.at[NB2 - 2]], ssem[3]).wait()
  pltpu.make_async_copy(rows[4], acc.at[dst_v.at[NB2 - 1]], ssem[4]).wait()
  plsc.subcore_barrier()

  @pl.when(s < NZT)
  def _():
    pltpu.sync_copy(acc.at[pl.ds(s * RPT, RPT)],
                    out.at[c, pl.ds(s * RPT, RPT)])


_BN = 1000
_NBK = N // _BN


def _m1_body(x_ref, w_ref, cntp_ref, z_ref, invc_ref):
  t = pl.program_id(0)
  i = pl.program_id(1)
  z = jnp.dot(x_ref[...], w_ref[0], preferred_element_type=_f32)
  z_ref[0, 0] = z[:, :CH]
  z_ref[1, 0] = z[:, CH:]

  @pl.when(jnp.logical_and(t == 0, i == 0))
  def _():
    csum = jnp.sum(cntp_ref[...], axis=0)
    invc_ref[...] = 1.0 / jnp.maximum(csum, 1.0)


def _z1_and_invc(x, w_r, cntp):
  return pl.pallas_call(
      _m1_body,
      grid=(NET, _NBK),
      in_specs=[
          pl.BlockSpec((_BN, C), lambda t, i: (i, 0)),
          pl.BlockSpec((1, C, C), lambda t, i: (t, 0, 0)),
          pl.BlockSpec((NW, CNT_SZ // C, C), lambda t, i: (0, 0, 0)),
      ],
      out_specs=[
          pl.BlockSpec((NC, 1, _BN, CH), lambda t, i: (0, t, i, 0)),
          pl.BlockSpec((CNT_SZ // C, C), lambda t, i: (0, 0)),
      ],
      out_shape=[
          jax.ShapeDtypeStruct((NC, NET, N, CH), _f32),
          jax.ShapeDtypeStruct((CNT_SZ // C, C), _f32),
      ],
  )(x, w_r, cntp)


def _bn1m2_body(s_ref, wb_ref, ga_ref, ba_ref, z2_ref, sum_ref, sq_ref):
  p = pl.program_id(0)
  i = pl.program_id(1)
  y = jnp.concatenate([s_ref[0], s_ref[1]], axis=-1)

  @pl.when(jnp.logical_and(p == 0, i == 0))
  def _():
    sum_ref[...] = jnp.zeros_like(sum_ref)
    sq_ref[...] = jnp.zeros_like(sq_ref)

  @pl.when(p == 0)
  def _():
    sum_ref[...] += jnp.sum(y, axis=0, keepdims=True)
    sq_ref[...] += jnp.sum(y * y, axis=0, keepdims=True)

  @pl.when(p == 1)
  def _():
    mean = sum_ref[...] * (1.0 / N)
    var = sq_ref[...] * (1.0 / N) - mean * mean
    inv = lax.rsqrt(var + EPS)
    x1 = jnp.maximum((y - mean) * inv * ga_ref[...] + ba_ref[...], 0.0)
    for t in range(NET):
      z = jnp.dot(x1, wb_ref[t], preferred_element_type=_f32)
      z2_ref[0, t] = z[:, :CH]
      z2_ref[1, t] = z[:, CH:]


def _bn1_then_z2(s1, wb_r, ga, ba):
  return pl.pallas_call(
      _bn1m2_body,
      grid=(2, _NBK),
      in_specs=[
          pl.BlockSpec((NC, _BN, CH), lambda p, i: (0, i, 0)),
          pl.BlockSpec((NET, C, C), lambda p, i: (0, 0, 0)),
          pl.BlockSpec((1, C), lambda p, i: (0, 0)),
          pl.BlockSpec((1, C), lambda p, i: (0, 0)),
      ],
      out_specs=pl.BlockSpec((NC, NET, _BN, CH), lambda p, i: (0, 0, i, 0)),
      out_shape=jax.ShapeDtypeStruct((NC, NET, N, CH), _f32),
      scratch_shapes=[
          pltpu.VMEM((1, C), _f32),
          pltpu.VMEM((1, C), _f32),
      ],
  )(s1, wb_r, ga, ba)


def _bn2_body(s_ref, x_ref, gb_ref, bb_ref, o_ref, sum_ref, sq_ref):
  p = pl.program_id(0)
  i = pl.program_id(1)
  y = jnp.concatenate([s_ref[0], s_ref[1]], axis=-1)

  @pl.when(jnp.logical_and(p == 0, i == 0))
  def _():
    sum_ref[...] = jnp.zeros_like(sum_ref)
    sq_ref[...] = jnp.zeros_like(sq_ref)

  @pl.when(p == 0)
  def _():
    sum_ref[...] += jnp.sum(y, axis=0, keepdims=True)
    sq_ref[...] += jnp.sum(y * y, axis=0, keepdims=True)

  @pl.when(p == 1)
  def _():
    mean = sum_ref[...] * (1.0 / N)
    var = sq_ref[...] * (1.0 / N) - mean * mean
    inv = lax.rsqrt(var + EPS)
    o_ref[...] = jnp.maximum(
        (y - mean) * inv * gb_ref[...] + bb_ref[...] + x_ref[...], 0.0)


def _bn2_skip(s2, x, gb, bb):
  return pl.pallas_call(
      _bn2_body,
      grid=(2, _NBK),
      in_specs=[
          pl.BlockSpec((NC, _BN, CH), lambda p, i: (0, i, 0)),
          pl.BlockSpec((_BN, C), lambda p, i: (i, 0)),
          pl.BlockSpec((1, C), lambda p, i: (0, 0)),
          pl.BlockSpec((1, C), lambda p, i: (0, 0)),
      ],
      out_specs=pl.BlockSpec((_BN, C), lambda p, i: (i, 0)),
      out_shape=jax.ShapeDtypeStruct((N, C), _f32),
      scratch_shapes=[
          pltpu.VMEM((1, C), _f32),
          pltpu.VMEM((1, C), _f32),
      ],
  )(s2, x, gb, bb)


def kernel(x, edge_index, edge_type, node_type, Wa, ga, ba, Wb, gb, bb):
  del node_type
  row = edge_index[0]
  col = edge_index[1]
  dst3 = row.reshape(NW, NB, B)
  col3 = col.reshape(NW, NB, B)
  et3 = edge_type.reshape(NW, NB, B)
  zeros2d = jnp.zeros((N, CH), _f32)
  zflat = jnp.zeros((CNT_SZ,), _f32)

  wa_r = Wa.reshape(NET, C, C)
  wb_r = Wb.reshape(NET, C, C)

  cntp = _count_kernel(dst3, et3, zflat)
  z1, invc = _z1_and_invc(x, wa_r, cntp.reshape(NW, CNT_SZ // C, C))
  scl3 = _scale_kernel(dst3, et3, invc.reshape(CNT_SZ))

  s1 = _conv_kernel(z1.reshape(NC * NET * N, CH), col3, et3, dst3, scl3,
                    zeros2d)
  z2 = _bn1_then_z2(s1, wb_r, ga.reshape(1, C), ba.reshape(1, C))
  s2 = _conv_kernel(z2.reshape(NC * NET * N, CH), col3, et3, dst3, scl3,
                    zeros2d)
  return _bn2_skip(s2, x, gb.reshape(1, C), bb.reshape(1, C))


# speedup vs baseline: 2.3625x; 1.2130x over previous
import functools

import jax
import jax.numpy as jnp
from jax import lax
from jax.experimental import pallas as pl
from jax.experimental.pallas import tpu as pltpu
from jax.experimental.pallas import tpu_sc as plsc

N = 10000
E = 320000
C = 128
NET = 7
EPS = 1e-5

NC = 2
NS = 16
NW = NC * NS
CH = C // NC
B = 80
NB = (E // NW) // B
NB2 = (E // NS) // B
G = B // 16
RPT = 1000
NZT = N // RPT
CNT_SZ = N * 8
CHK = 25

_mesh = plsc.VectorSubcoreMesh(core_axis_name="c", subcore_axis_name="s")

_f32 = jnp.float32
_i32 = jnp.int32


@functools.partial(
    pl.kernel,
    out_type=jax.ShapeDtypeStruct((NW, CNT_SZ), _f32),
    mesh=_mesh,
    compiler_params=pltpu.CompilerParams(needs_layout_passes=False),
    scratch_types=[
        pltpu.VMEM((NB, B), _i32),
        pltpu.VMEM((NB, B), _i32),
        pltpu.VMEM((CNT_SZ,), _f32),
    ],
)
def _count_kernel(dst3, et3, zflat, cntp, dst_v, et_v, cnt_v):
  c = lax.axis_index("c")
  s = lax.axis_index("s")
  wid = s * NC + c
  pltpu.sync_copy(dst3.at[wid], dst_v)
  pltpu.sync_copy(et3.at[wid], et_v)
  pltpu.sync_copy(zflat.at[pl.ds(0, CNT_SZ)], cnt_v)
  ones = jnp.ones((16,), _f32)

  def body(b, carry):
    for g in range(G):
      r = dst_v[b, pl.ds(16 * g, 16)]
      e = et_v[b, pl.ds(16 * g, 16)]
      plsc.addupdate_scatter(cnt_v, [r * 8 + e], ones)
    return carry

  lax.fori_loop(0, NB, body, 0)
  pltpu.sync_copy(cnt_v, cntp.at[wid])


@functools.partial(
    pl.kernel,
    out_type=jax.ShapeDtypeStruct((NW, NB, B), _f32),
    mesh=_mesh,
    compiler_params=pltpu.CompilerParams(needs_layout_passes=False),
    scratch_types=[
        pltpu.VMEM((NB, B), _i32),
        pltpu.VMEM((NB, B), _i32),
        pltpu.VMEM((CNT_SZ,), _f32),
        pltpu.VMEM((NB, B), _f32),
    ],
)
def _scale_kernel(dst3, et3, invc, scl3, dst_v, et_v, invc_v, scl_v):
  c = lax.axis_index("c")
  s = lax.axis_index("s")
  wid = s * NC + c
  pltpu.sync_copy(dst3.at[wid], dst_v)
  pltpu.sync_copy(et3.at[wid], et_v)
  pltpu.sync_copy(invc, invc_v)

  def body(b, carry):
    for g in range(G):
      r = dst_v[b, pl.ds(16 * g, 16)]
      e = et_v[b, pl.ds(16 * g, 16)]
      scl_v[b, pl.ds(16 * g, 16)] = plsc.load_gather(invc_v, [r * 8 + e])
    return carry

  lax.fori_loop(0, NB, body, 0)
  pltpu.sync_copy(scl_v, scl3.at[wid])


@functools.partial(
    pl.kernel,
    out_type=jax.ShapeDtypeStruct((NC, N, CH), _f32),
    mesh=_mesh,
    compiler_params=pltpu.CompilerParams(
        needs_layout_passes=False, use_tc_tiling_on_sc=False),
    scratch_types=[
        pltpu.VMEM((NB2, B), _i32),
        pltpu.VMEM((NB2, B), _i32),
        pltpu.VMEM((NB2, B), _f32),
        pltpu.VMEM((CHK, B), _i32),
        pltpu.VMEM((CHK, B), _i32),
        pltpu.VMEM((B, CH), _f32),
        pltpu.VMEM((B, CH), _f32),
        pltpu.VMEM((B, CH), _f32),
        pltpu.VMEM((B, CH), _f32),
        pltpu.VMEM((B, CH), _f32),
        pltpu.VMEM_SHARED((N, CH), _f32),
        pltpu.SemaphoreType.DMA,
        pltpu.SemaphoreType.DMA,
        pltpu.SemaphoreType.DMA,
        pltpu.SemaphoreType.DMA,
        pltpu.SemaphoreType.DMA,
        pltpu.SemaphoreType.DMA,
        pltpu.SemaphoreType.DMA,
        pltpu.SemaphoreType.DMA,
        pltpu.SemaphoreType.DMA,
        pltpu.SemaphoreType.DMA,
    ],
)
def _conv_kernel(zh, col3, et3, dst3, scl3, zeros2d, out,
                 zarr, dst_v, scl_v, ctmp, etmp,
                 r0, r1, r2, r3, r4, acc,
                 g0, g1, g2, g3, g4, s0, s1, s2, s3, s4):
  c = lax.axis_index("c")
  s = lax.axis_index("s")

  @pl.when(s < NZT)
  def _():
    pltpu.sync_copy(zeros2d.at[pl.ds(s * RPT, RPT)],
                    acc.at[pl.ds(s * RPT, RPT)])

  for h in range(2):
    pltpu.sync_copy(dst3.at[2 * s + h], dst_v.at[pl.ds(h * NB, NB)])
    pltpu.sync_copy(scl3.at[2 * s + h], scl_v.at[pl.ds(h * NB, NB)])
    for cc in range(NB // CHK):
      pltpu.sync_copy(col3.at[2 * s + h, pl.ds(cc * CHK, CHK)], ctmp)
      pltpu.sync_copy(et3.at[2 * s + h, pl.ds(cc * CHK, CHK)], etmp)
      base_b = h * NB + cc * CHK

      def zbody(g2, carry):
        for g in range(G):
          zarr[base_b + g2, pl.ds(16 * g, 16)] = (
              ctmp[g2, pl.ds(16 * g, 16)] * 14
              + etmp[g2, pl.ds(16 * g, 16)] * 2 + c)
        return carry

      lax.fori_loop(0, CHK, zbody, 0)
  plsc.subcore_barrier()

  rows = [r0, r1, r2, r3, r4]
  gsem = [g0, g1, g2, g3, g4]
  ssem = [s0, s1, s2, s3, s4]
  NBUF = 5

  def scale_rows(b, rbuf):
    bvec = jnp.full((16,), b, _i32)

    @plsc.parallel_loop(0, B, 1, unroll=8)
    def _(i):
      bc = plsc.load_gather(scl_v, [bvec, jnp.full((16,), i, _i32)])
      for k in range(CH // 16):
        rbuf[i, pl.ds(16 * k, 16)] = rbuf[i, pl.ds(16 * k, 16)] * bc

  for q in range(3):
    pltpu.async_copy(zh.at[zarr.at[q]], rows[q], gsem[q])

  def round_body(j, carry):
    for q in range(NBUF):
      b = NBUF * j + q
      qn = (q + 3) % NBUF
      pltpu.make_async_copy(zh.at[zarr.at[b]], rows[q], gsem[q]).wait()
      scale_rows(b, rows[q])
      pltpu.async_copy(rows[q], acc.at[dst_v.at[b]], ssem[q], add=True)

      @pl.when(b >= 2)
      def _(qn=qn, b=b):
        pltpu.make_async_copy(
            rows[qn], acc.at[dst_v.at[b - 2]], ssem[qn]).wait()

      @pl.when(b + 3 < NB2)
      def _(qn=qn, b=b):
        pltpu.async_copy(zh.at[zarr.at[b + 3]], rows[qn], gsem[qn])

    return carry

  lax.fori_loop(0, NB2 // NBUF, round_body, 0)
  pltpu.make_async_copy(rows[3], acc.at[dst_v.at[NB2 - 2]], ssem[3]).wait()
  pltpu.make_async_copy(rows[4], acc.at[dst_v.at[NB2 - 1]], ssem[4]).wait()
  plsc.subcore_barrier()

  @pl.when(s < NZT)
  def _():
    pltpu.sync_copy(acc.at[pl.ds(s * RPT, RPT)],
                    out.at[c, pl.ds(s * RPT, RPT)])


_BN = 1000
_NBK = N // _BN


def _m1_body(x_ref, w_ref, cntp_ref, z_ref, invc_ref):
  i = pl.program_id(0)
  z_ref[...] = jnp.dot(x_ref[...], w_ref[...], preferred_element_type=_f32)

  @pl.when(i == 0)
  def _():
    csum = jnp.sum(cntp_ref[...], axis=0)
    invc_ref[...] = 1.0 / jnp.maximum(csum, 1.0)


def _z1_and_invc(x_bf, wcat, cntp):
  return pl.pallas_call(
      _m1_body,
      grid=(_NBK,),
      in_specs=[
          pl.BlockSpec((_BN, C), lambda i: (i, 0)),
          pl.BlockSpec((C, NET * C), lambda i: (0, 0)),
          pl.BlockSpec((NW, CNT_SZ // C, C), lambda i: (0, 0, 0)),
      ],
      out_specs=[
          pl.BlockSpec((_BN, NET * C), lambda i: (i, 0)),
          pl.BlockSpec((CNT_SZ // C, C), lambda i: (0, 0)),
      ],
      out_shape=[
          jax.ShapeDtypeStruct((N, NET * C), _f32),
          jax.ShapeDtypeStruct((CNT_SZ // C, C), _f32),
      ],
  )(x_bf, wcat, cntp)


def _bn1m2_body(s_ref, wb_ref, ga_ref, ba_ref, z2_ref, sum_ref, sq_ref):
  p = pl.program_id(0)
  i = pl.program_id(1)
  y = jnp.concatenate([s_ref[0], s_ref[1]], axis=-1)

  @pl.when(jnp.logical_and(p == 0, i == 0))
  def _():
    sum_ref[...] = jnp.zeros_like(sum_ref)
    sq_ref[...] = jnp.zeros_like(sq_ref)

  @pl.when(p == 0)
  def _():
    sum_ref[...] += jnp.sum(y, axis=0, keepdims=True)
    sq_ref[...] += jnp.sum(y * y, axis=0, keepdims=True)

  @pl.when(p == 1)
  def _():
    mean = sum_ref[...] * (1.0 / N)
    var = sq_ref[...] * (1.0 / N) - mean * mean
    inv = lax.rsqrt(var + EPS)
    x1 = jnp.maximum((y - mean) * inv * ga_ref[...] + ba_ref[...], 0.0)
    z2_ref[...] = jnp.dot(x1.astype(jnp.bfloat16), wb_ref[...],
                          preferred_element_type=_f32)


def _bn1_then_z2(s1, wcat_b, ga, ba):
  return pl.pallas_call(
      _bn1m2_body,
      grid=(2, _NBK),
      in_specs=[
          pl.BlockSpec((NC, _BN, CH), lambda p, i: (0, i, 0)),
          pl.BlockSpec((C, NET * C), lambda p, i: (0, 0)),
          pl.BlockSpec((1, C), lambda p, i: (0, 0)),
          pl.BlockSpec((1, C), lambda p, i: (0, 0)),
      ],
      out_specs=pl.BlockSpec((_BN, NET * C), lambda p, i: (i, 0)),
      out_shape=jax.ShapeDtypeStruct((N, NET * C), _f32),
      scratch_shapes=[
          pltpu.VMEM((1, C), _f32),
          pltpu.VMEM((1, C), _f32),
      ],
  )(s1, wcat_b, ga, ba)


def _bn2_body(s_ref, x_ref, gb_ref, bb_ref, o_ref, sum_ref, sq_ref):
  p = pl.program_id(0)
  i = pl.program_id(1)
  y = jnp.concatenate([s_ref[0], s_ref[1]], axis=-1)

  @pl.when(jnp.logical_and(p == 0, i == 0))
  def _():
    sum_ref[...] = jnp.zeros_like(sum_ref)
    sq_ref[...] = jnp.zeros_like(sq_ref)

  @pl.when(p == 0)
  def _():
    sum_ref[...] += jnp.sum(y, axis=0, keepdims=True)
    sq_ref[...] += jnp.sum(y * y, axis=0, keepdims=True)

  @pl.when(p == 1)
  def _():
    mean = sum_ref[...] * (1.0 / N)
    var = sq_ref[...] * (1.0 / N) - mean * mean
    inv = lax.rsqrt(var + EPS)
    o_ref[...] = jnp.maximum(
        (y - mean) * inv * gb_ref[...] + bb_ref[...] + x_ref[...], 0.0)


def _bn2_skip(s2, x, gb, bb):
  return pl.pallas_call(
      _bn2_body,
      grid=(2, _NBK),
      in_specs=[
          pl.BlockSpec((NC, _BN, CH), lambda p, i: (0, i, 0)),
          pl.BlockSpec((_BN, C), lambda p, i: (i, 0)),
          pl.BlockSpec((1, C), lambda p, i: (0, 0)),
          pl.BlockSpec((1, C), lambda p, i: (0, 0)),
      ],
      out_specs=pl.BlockSpec((_BN, C), lambda p, i: (i, 0)),
      out_shape=jax.ShapeDtypeStruct((N, C), _f32),
      scratch_shapes=[
          pltpu.VMEM((1, C), _f32),
          pltpu.VMEM((1, C), _f32),
      ],
  )(s2, x, gb, bb)


def kernel(x, edge_index, edge_type, node_type, Wa, ga, ba, Wb, gb, bb):
  del node_type
  row = edge_index[0]
  col = edge_index[1]
  dst3 = row.reshape(NW, NB, B)
  col3 = col.reshape(NW, NB, B)
  et3 = edge_type.reshape(NW, NB, B)
  zeros2d = jnp.zeros((N, CH), _f32)
  zflat = jnp.zeros((CNT_SZ,), _f32)

  wcat_a = Wa.reshape(NET, C, C).transpose(1, 0, 2).reshape(C, NET * C)
  wcat_b = Wb.reshape(NET, C, C).transpose(1, 0, 2).reshape(C, NET * C)
  wcat_a = wcat_a.astype(jnp.bfloat16)
  wcat_b = wcat_b.astype(jnp.bfloat16)

  cntp = _count_kernel(dst3, et3, zflat)
  z1, invc = _z1_and_invc(x.astype(jnp.bfloat16), wcat_a,
                          cntp.reshape(NW, CNT_SZ // C, C))
  scl3 = _scale_kernel(dst3, et3, invc.reshape(CNT_SZ))

  s1 = _conv_kernel(z1.reshape(N * 2 * NET, CH), col3, et3, dst3, scl3,
                    zeros2d)
  z2 = _bn1_then_z2(s1, wcat_b, ga.reshape(1, C), ba.reshape(1, C))
  s2 = _conv_kernel(z2.reshape(N * 2 * NET, CH), col3, et3, dst3, scl3,
                    zeros2d)
  return _bn2_skip(s2, x, gb.reshape(1, C), bb.reshape(1, C))


# speedup vs baseline: 2.4166x; 1.0229x over previous
import functools

import jax
import jax.numpy as jnp
from jax import lax
from jax.experimental import pallas as pl
from jax.experimental.pallas import tpu as pltpu
from jax.experimental.pallas import tpu_sc as plsc

N = 10000
E = 320000
C = 128
NET = 7
EPS = 1e-5

NC = 2
NS = 16
NW = NC * NS
CH = C // NC
B = 80
NB = (E // NW) // B
NB2 = (E // NS) // B
G = B // 16
RPT = 1000
NZT = N // RPT
CNT_SZ = N * 8
CHK = 25

_mesh = plsc.VectorSubcoreMesh(core_axis_name="c", subcore_axis_name="s")

_f32 = jnp.float32
_i32 = jnp.int32


@functools.partial(
    pl.kernel,
    out_type=jax.ShapeDtypeStruct((NW, CNT_SZ), _f32),
    mesh=_mesh,
    compiler_params=pltpu.CompilerParams(needs_layout_passes=False),
    scratch_types=[
        pltpu.VMEM((NB, B), _i32),
        pltpu.VMEM((NB, B), _i32),
        pltpu.VMEM((CNT_SZ,), _f32),
    ],
)
def _count_kernel(dst3, et3, zflat, cntp, dst_v, et_v, cnt_v):
  c = lax.axis_index("c")
  s = lax.axis_index("s")
  wid = s * NC + c
  pltpu.sync_copy(dst3.at[wid], dst_v)
  pltpu.sync_copy(et3.at[wid], et_v)
  pltpu.sync_copy(zflat.at[pl.ds(0, CNT_SZ)], cnt_v)
  ones = jnp.ones((16,), _f32)

  def body(b, carry):
    for g in range(G):
      r = dst_v[b, pl.ds(16 * g, 16)]
      e = et_v[b, pl.ds(16 * g, 16)]
      plsc.addupdate_scatter(cnt_v, [r * 8 + e], ones)
    return carry

  lax.fori_loop(0, NB, body, 0)
  pltpu.sync_copy(cnt_v, cntp.at[wid])


@functools.partial(
    pl.kernel,
    out_type=jax.ShapeDtypeStruct((NW, NB, B), _f32),
    mesh=_mesh,
    compiler_params=pltpu.CompilerParams(needs_layout_passes=False),
    scratch_types=[
        pltpu.VMEM((NB, B), _i32),
        pltpu.VMEM((NB, B), _i32),
        pltpu.VMEM((CNT_SZ,), _f32),
        pltpu.VMEM((NB, B), _f32),
    ],
)
def _scale_kernel(dst3, et3, invc, scl3, dst_v, et_v, invc_v, scl_v):
  c = lax.axis_index("c")
  s = lax.axis_index("s")
  wid = s * NC + c
  pltpu.sync_copy(dst3.at[wid], dst_v)
  pltpu.sync_copy(et3.at[wid], et_v)
  pltpu.sync_copy(invc, invc_v)

  def body(b, carry):
    for g in range(G):
      r = dst_v[b, pl.ds(16 * g, 16)]
      e = et_v[b, pl.ds(16 * g, 16)]
      scl_v[b, pl.ds(16 * g, 16)] = plsc.load_gather(invc_v, [r * 8 + e])
    return carry

  lax.fori_loop(0, NB, body, 0)
  pltpu.sync_copy(scl_v, scl3.at[wid])


@functools.partial(
    pl.kernel,
    out_type=jax.ShapeDtypeStruct((NC, N, CH), _f32),
    mesh=_mesh,
    compiler_params=pltpu.CompilerParams(
        needs_layout_passes=False, use_tc_tiling_on_sc=False),
    scratch_types=[
        pltpu.VMEM((NB2, B), _i32),
        pltpu.VMEM((NB2, B), _i32),
        pltpu.VMEM((NB2, B), _f32),
        pltpu.VMEM((CHK, B), _i32),
        pltpu.VMEM((CHK, B), _i32),
        pltpu.VMEM((B, CH), _f32),
        pltpu.VMEM((B, CH), _f32),
        pltpu.VMEM((B, CH), _f32),
        pltpu.VMEM((B, CH), _f32),
        pltpu.VMEM((B, CH), _f32),
        pltpu.VMEM_SHARED((N, CH), _f32),
        pltpu.SemaphoreType.DMA,
        pltpu.SemaphoreType.DMA,
        pltpu.SemaphoreType.DMA,
        pltpu.SemaphoreType.DMA,
        pltpu.SemaphoreType.DMA,
        pltpu.SemaphoreType.DMA,
        pltpu.SemaphoreType.DMA,
        pltpu.SemaphoreType.DMA,
        pltpu.SemaphoreType.DMA,
        pltpu.SemaphoreType.DMA,
    ],
)
def _conv_kernel(zh, col3, et3, dst3, scl3, zeros2d, out,
                 zarr, dst_v, scl_v, ctmp, etmp,
                 r0, r1, r2, r3, r4, acc,
                 g0, g1, g2, g3, g4, s0, s1, s2, s3, s4):
  c = lax.axis_index("c")
  s = lax.axis_index("s")

  @pl.when(s < NZT)
  def _():
    pltpu.sync_copy(zeros2d.at[pl.ds(s * RPT, RPT)],
                    acc.at[pl.ds(s * RPT, RPT)])

  for h in range(2):
    pltpu.sync_copy(dst3.at[2 * s + h], dst_v.at[pl.ds(h * NB, NB)])
    pltpu.sync_copy(scl3.at[2 * s + h], scl_v.at[pl.ds(h * NB, NB)])
    for cc in range(NB // CHK):
      pltpu.sync_copy(col3.at[2 * s + h, pl.ds(cc * CHK, CHK)], ctmp)
      pltpu.sync_copy(et3.at[2 * s + h, pl.ds(cc * CHK, CHK)], etmp)
      base_b = h * NB + cc * CHK

      def zbody(g2, carry):
        for g in range(G):
          zarr[base_b + g2, pl.ds(16 * g, 16)] = (
              ctmp[g2, pl.ds(16 * g, 16)] * 14
              + etmp[g2, pl.ds(16 * g, 16)] * 2 + c)
        return carry

      lax.fori_loop(0, CHK, zbody, 0)
  plsc.subcore_barrier()

  rows = [r0, r1, r2, r3, r4]
  gsem = [g0, g1, g2, g3, g4]
  ssem = [s0, s1, s2, s3, s4]
  NBUF = 5

  def scale_rows(b, rbuf):
    bvec = jnp.full((16,), b, _i32)

    @plsc.parallel_loop(0, B, 1, unroll=8)
    def _(i):
      bc = plsc.load_gather(scl_v, [bvec, jnp.full((16,), i, _i32)])
      for k in range(CH // 16):
        rbuf[i, pl.ds(16 * k, 16)] = rbuf[i, pl.ds(16 * k, 16)] * bc

  for q in range(3):
    pltpu.async_copy(zh.at[zarr.at[q]], rows[q], gsem[q])

  def round_body(j, carry):
    for q in range(NBUF):
      b = NBUF * j + q
      qn = (q + 3) % NBUF
      pltpu.make_async_copy(zh.at[zarr.at[b]], rows[q], gsem[q]).wait()
      scale_rows(b, rows[q])
      pltpu.async_copy(rows[q], acc.at[dst_v.at[b]], ssem[q], add=True)

      @pl.when(b >= 2)
      def _(qn=qn, b=b):
        pltpu.make_async_copy(
            rows[qn], acc.at[dst_v.at[b - 2]], ssem[qn]).wait()

      @pl.when(b + 3 < NB2)
      def _(qn=qn, b=b):
        pltpu.async_copy(zh.at[zarr.at[b + 3]], rows[qn], gsem[qn])

    return carry

  lax.fori_loop(0, NB2 // NBUF, round_body, 0)
  pltpu.make_async_copy(rows[3], acc.at[dst_v.at[NB2 - 2]], ssem[3]).wait()
  pltpu.make_async_copy(rows[4], acc.at[dst_v.at[NB2 - 1]], ssem[4]).wait()
  plsc.subcore_barrier()

  @pl.when(s < NZT)
  def _():
    pltpu.sync_copy(acc.at[pl.ds(s * RPT, RPT)],
                    out.at[c, pl.ds(s * RPT, RPT)])


_BN = 2000
_NBK = N // _BN


def _m1_body(x_ref, w_ref, cntp_ref, z_ref, invc_ref):
  i = pl.program_id(0)
  z_ref[...] = jnp.dot(x_ref[...], w_ref[...], preferred_element_type=_f32)

  @pl.when(i == 0)
  def _():
    csum = jnp.sum(cntp_ref[...], axis=0)
    invc_ref[...] = 1.0 / jnp.maximum(csum, 1.0)


def _z1_and_invc(x_bf, wcat, cntp):
  return pl.pallas_call(
      _m1_body,
      grid=(_NBK,),
      in_specs=[
          pl.BlockSpec((_BN, C), lambda i: (i, 0)),
          pl.BlockSpec((C, NET * C), lambda i: (0, 0)),
          pl.BlockSpec((NW, CNT_SZ // C, C), lambda i: (0, 0, 0)),
      ],
      out_specs=[
          pl.BlockSpec((_BN, NET * C), lambda i: (i, 0)),
          pl.BlockSpec((CNT_SZ // C, C), lambda i: (0, 0)),
      ],
      out_shape=[
          jax.ShapeDtypeStruct((N, NET * C), _f32),
          jax.ShapeDtypeStruct((CNT_SZ // C, C), _f32),
      ],
  )(x_bf, wcat, cntp)


def _bn1m2_body(s_ref, wb_ref, ga_ref, ba_ref, z2_ref, sum_ref, sq_ref):
  p = pl.program_id(0)
  i = pl.program_id(1)
  y = jnp.concatenate([s_ref[0], s_ref[1]], axis=-1)

  @pl.when(jnp.logical_and(p == 0, i == 0))
  def _():
    sum_ref[...] = jnp.zeros_like(sum_ref)
    sq_ref[...] = jnp.zeros_like(sq_ref)

  @pl.when(p == 0)
  def _():
    sum_ref[...] += jnp.sum(y, axis=0, keepdims=True)
    sq_ref[...] += jnp.sum(y * y, axis=0, keepdims=True)

  @pl.when(p == 1)
  def _():
    mean = sum_ref[...] * (1.0 / N)
    var = sq_ref[...] * (1.0 / N) - mean * mean
    inv = lax.rsqrt(var + EPS)
    x1 = jnp.maximum((y - mean) * inv * ga_ref[...] + ba_ref[...], 0.0)
    z2_ref[...] = jnp.dot(x1.astype(jnp.bfloat16), wb_ref[...],
                          preferred_element_type=_f32)


def _bn1_then_z2(s1, wcat_b, ga, ba):
  return pl.pallas_call(
      _bn1m2_body,
      grid=(2, _NBK),
      in_specs=[
          pl.BlockSpec((NC, _BN, CH), lambda p, i: (0, i, 0)),
          pl.BlockSpec((C, NET * C), lambda p, i: (0, 0)),
          pl.BlockSpec((1, C), lambda p, i: (0, 0)),
          pl.BlockSpec((1, C), lambda p, i: (0, 0)),
      ],
      out_specs=pl.BlockSpec((_BN, NET * C), lambda p, i: (i, 0)),
      out_shape=jax.ShapeDtypeStruct((N, NET * C), _f32),
      scratch_shapes=[
          pltpu.VMEM((1, C), _f32),
          pltpu.VMEM((1, C), _f32),
      ],
  )(s1, wcat_b, ga, ba)


def _bn2_body(s_ref, x_ref, gb_ref, bb_ref, o_ref, sum_ref, sq_ref):
  p = pl.program_id(0)
  i = pl.program_id(1)
  y = jnp.concatenate([s_ref[0], s_ref[1]], axis=-1)

  @pl.when(jnp.logical_and(p == 0, i == 0))
  def _():
    sum_ref[...] = jnp.zeros_like(sum_ref)
    sq_ref[...] = jnp.zeros_like(sq_ref)

  @pl.when(p == 0)
  def _():
    sum_ref[...] += jnp.sum(y, axis=0, keepdims=True)
    sq_ref[...] += jnp.sum(y * y, axis=0, keepdims=True)

  @pl.when(p == 1)
  def _():
    mean = sum_ref[...] * (1.0 / N)
    var = sq_ref[...] * (1.0 / N) - mean * mean
    inv = lax.rsqrt(var + EPS)
    o_ref[...] = jnp.maximum(
        (y - mean) * inv * gb_ref[...] + bb_ref[...] + x_ref[...], 0.0)


def _bn2_skip(s2, x, gb, bb):
  return pl.pallas_call(
      _bn2_body,
      grid=(2, _NBK),
      in_specs=[
          pl.BlockSpec((NC, _BN, CH), lambda p, i: (0, i, 0)),
          pl.BlockSpec((_BN, C), lambda p, i: (i, 0)),
          pl.BlockSpec((1, C), lambda p, i: (0, 0)),
          pl.BlockSpec((1, C), lambda p, i: (0, 0)),
      ],
      out_specs=pl.BlockSpec((_BN, C), lambda p, i: (i, 0)),
      out_shape=jax.ShapeDtypeStruct((N, C), _f32),
      scratch_shapes=[
          pltpu.VMEM((1, C), _f32),
          pltpu.VMEM((1, C), _f32),
      ],
  )(s2, x, gb, bb)


def kernel(x, edge_index, edge_type, node_type, Wa, ga, ba, Wb, gb, bb):
  del node_type
  row = edge_index[0]
  col = edge_index[1]
  dst3 = row.reshape(NW, NB, B)
  col3 = col.reshape(NW, NB, B)
  et3 = edge_type.reshape(NW, NB, B)
  zeros2d = jnp.zeros((N, CH), _f32)
  zflat = jnp.zeros((CNT_SZ,), _f32)

  wcat_a = Wa.reshape(NET, C, C).transpose(1, 0, 2).reshape(C, NET * C)
  wcat_b = Wb.reshape(NET, C, C).transpose(1, 0, 2).reshape(C, NET * C)
  wcat_a = wcat_a.astype(jnp.bfloat16)
  wcat_b = wcat_b.astype(jnp.bfloat16)

  cntp = _count_kernel(dst3, et3, zflat)
  z1, invc = _z1_and_invc(x.astype(jnp.bfloat16), wcat_a,
                          cntp.reshape(NW, CNT_SZ // C, C))
  scl3 = _scale_kernel(dst3, et3, invc.reshape(CNT_SZ))

  s1 = _conv_kernel(z1.reshape(N * 2 * NET, CH), col3, et3, dst3, scl3,
                    zeros2d)
  z2 = _bn1_then_z2(s1, wcat_b, ga.reshape(1, C), ba.reshape(1, C))
  s2 = _conv_kernel(z2.reshape(N * 2 * NET, CH), col3, et3, dst3, scl3,
                    zeros2d)
  return _bn2_skip(s2, x, gb.reshape(1, C), bb.reshape(1, C))
